# baseline (speedup 1.0000x reference)
import jax
import jax.numpy as jnp
from jax import lax
from jax.experimental import pallas as pl
from jax.experimental.pallas import tpu as pltpu
from jax.experimental.pallas import tpu_sc as plsc

_NC = 2
_NS = 16
_NW = _NC * _NS
_C = 16
_L = 16
_B = 4
_E = 768


def _add_chunk(dst, src):
    @plsc.parallel_loop(0, _C * _E // _L, 1, unroll=8)
    def _(i):
        sl = pl.ds(i * _L, _L)
        dst[sl] = dst[sl] + src[sl]


def _pe_body(x_hbm, t_hbm, o_hbm, tbufs, xbufs, sem_t, sems_x, sems_o):
    n_seq = t_hbm.shape[0] // _E
    se = n_seq * _E
    cw = _C * _E
    seq_per_w = n_seq // _NW
    n_chunks = seq_per_w // _C
    wid = lax.axis_index("s") * _NC + lax.axis_index("c")
    seq0 = wid * seq_per_w

    def chunk_off(g):
        return (seq0 + g * _C) * _E

    def start_loads(g, slot):
        s = chunk_off(g)
        pltpu.make_async_copy(t_hbm.at[pl.ds(s, cw)], tbufs[slot],
                              sem_t[slot]).start()
        for b in range(_B):
            pltpu.make_async_copy(x_hbm.at[pl.ds(b * se + s, cw)],
                                  xbufs[slot][b], sems_x[slot][b]).start()

    def wait_loads(g, slot):
        s = chunk_off(g)
        pltpu.make_async_copy(t_hbm.at[pl.ds(s, cw)], tbufs[slot],
                              sem_t[slot]).wait()
        for b in range(_B):
            pltpu.make_async_copy(x_hbm.at[pl.ds(b * se + s, cw)],
                                  xbufs[slot][b], sems_x[slot][b]).wait()

    def wait_stores(g, slot):
        s = chunk_off(g)
        for b in range(_B):
            pltpu.make_async_copy(xbufs[slot][b],
                                  o_hbm.at[pl.ds(b * se + s, cw)],
                                  sems_o[slot][b]).wait()

    start_loads(0, 0)

    def step(go, carry):
        for slot in range(2):
            g = 2 * go + slot
            ns = 1 - slot

            @pl.when(g >= 1)
            def _():
                wait_stores(g - 1, ns)

            @pl.when(g <= n_chunks - 2)
            def _():
                start_loads(g + 1, ns)

            wait_loads(g, slot)
            s = chunk_off(g)
            for b in range(_B):
                _add_chunk(xbufs[slot][b], tbufs[slot])
                pltpu.make_async_copy(xbufs[slot][b],
                                      o_hbm.at[pl.ds(b * se + s, cw)],
                                      sems_o[slot][b]).start()
        return carry

    lax.fori_loop(0, n_chunks // 2, step, 0)
    wait_stores(n_chunks - 1, (n_chunks - 1) % 2)


def kernel(x, pos_table):
    B, S, E = x.shape
    x2 = x.reshape(B * S * E)
    t2 = pos_table.reshape(S * E)
    mesh = plsc.VectorSubcoreMesh(
        core_axis_name="c", subcore_axis_name="s",
        num_cores=_NC, num_subcores=_NS,
    )
    out = pl.kernel(
        _pe_body,
        out_type=jax.ShapeDtypeStruct((B * S * E,), x.dtype),
        mesh=mesh,
        scratch_types=[
            [pltpu.VMEM((_C * E,), jnp.float32) for _ in range(2)],
            [[pltpu.VMEM((_C * E,), jnp.float32) for _ in range(_B)]
             for _ in range(2)],
            [pltpu.SemaphoreType.DMA for _ in range(2)],
            [[pltpu.SemaphoreType.DMA for _ in range(_B)] for _ in range(2)],
            [[pltpu.SemaphoreType.DMA for _ in range(_B)] for _ in range(2)],
        ],
    )(x2, t2)
    return out.reshape(B, S, E)


# speedup vs baseline: 4.6475x; 4.6475x over previous
import jax
import jax.numpy as jnp
from jax.experimental import pallas as pl

_BS = 1024


def _add_kernel(x_ref, t_ref, o_ref):
    o_ref[...] = x_ref[...] + t_ref[...][None, :, :]


def kernel(x, pos_table):
    B, S, E = x.shape
    return pl.pallas_call(
        _add_kernel,
        grid=(S // _BS,),
        in_specs=[
            pl.BlockSpec((B, _BS, E), lambda j: (0, j, 0)),
            pl.BlockSpec((_BS, E), lambda j: (j, 0)),
        ],
        out_specs=pl.BlockSpec((B, _BS, E), lambda j: (0, j, 0)),
        out_shape=jax.ShapeDtypeStruct((B, S, E), x.dtype),
    )(x, pos_table)
